# Initial kernel scaffold; baseline (speedup 1.0000x reference)
import jax
import jax.numpy as jnp
from jax.experimental import pallas as pl


def kernel(logits, logit_lengths):
    raise NotImplementedError("write your pallas kernel here")


# speedup vs baseline: 4.1305x; 4.1305x over previous
import functools

import jax
import jax.numpy as jnp
import numpy as np
from jax.experimental import pallas as pl
from jax.experimental.pallas import tpu as pltpu

K = 40
V = 128
TB = 512


def _bitonic_sort_desc(keys):
    rows = keys.shape[0]
    lane = jax.lax.broadcasted_iota(jnp.int32, (rows, V), 1)
    for k in (2, 4, 8, 16, 32, 64, 128):
        j = k // 2
        while j >= 1:
            down = pltpu.roll(keys, V - j, 1)
            up = pltpu.roll(keys, j, 1)
            is_lower = (lane & j) == 0
            partner = jnp.where(is_lower, down, up)
            seg_desc = (lane & k) == 0
            keep_max = seg_desc == is_lower
            keys = jnp.where(
                keep_max, jnp.maximum(keys, partner), jnp.minimum(keys, partner)
            )
            j //= 2
    return keys


def _ctc_kernel(len_ref, x_ref, probs_ref, packed_ref, *, tb):
    n = pl.program_id(0)
    t0 = pl.program_id(1) * tb
    x = x_ref[...]

    m = jnp.max(x, axis=-1, keepdims=True)
    e = jnp.exp(x - m)
    s = jnp.sum(e, axis=-1, keepdims=True)
    p = e * (1.0 / s)

    lane = jax.lax.broadcasted_iota(jnp.int32, (tb, V), 1)
    pb = jax.lax.bitcast_convert_type(p, jnp.int32)
    key = ((pb + 64) & jnp.int32(~127)) | (127 - lane)

    skey = _bitonic_sort_desc(key)[:, :K]
    vals = jax.lax.bitcast_convert_type(skey & jnp.int32(~127), jnp.float32)
    idx = 127 - (skey & 127)

    t_row = t0 + jax.lax.broadcasted_iota(jnp.int32, (tb, 1), 0)
    mask = t_row < len_ref[n]

    probs_ref[0, :, :] = jnp.where(mask, vals, 0.0)

    selr = jax.lax.broadcasted_iota(jnp.int32, (K, 3 * K), 0)
    selc = jax.lax.broadcasted_iota(jnp.int32, (K, 3 * K), 1)
    sel = (selc == 3 * selr + 2).astype(jnp.float32)
    idx_rep = jax.lax.dot_general(
        idx.astype(jnp.float32),
        sel,
        (((1,), (0,)), ((), ())),
        preferred_element_type=jnp.float32,
    ).astype(jnp.int32)
    lane3 = jax.lax.broadcasted_iota(jnp.int32, (tb, 3 * K), 1) % 3
    interleaved = jnp.where(
        lane3 == 0, n, jnp.where(lane3 == 1, t_row, idx_rep)
    )
    packed_ref[0, :, :] = jnp.where(mask, interleaved, -1)


@jax.jit
def kernel(logits, logit_lengths):
    T, N, _ = logits.shape
    tb = TB
    grid = (N, T // tb)
    probs, packed = pl.pallas_call(
        functools.partial(_ctc_kernel, tb=tb),
        grid=grid,
        in_specs=[
            pl.BlockSpec(memory_space=pltpu.SMEM),
            pl.BlockSpec((tb, V), lambda n, t: (t, n)),
        ],
        out_specs=[
            pl.BlockSpec((1, tb, K), lambda n, t: (n, t, 0)),
            pl.BlockSpec((1, tb, 3 * K), lambda n, t: (n, t, 0)),
        ],
        out_shape=[
            jax.ShapeDtypeStruct((N, T, K), jnp.float32),
            jax.ShapeDtypeStruct((N, T, 3 * K), jnp.int32),
        ],
        compiler_params=pltpu.CompilerParams(
            dimension_semantics=("parallel", "parallel"),
        ),
    )(logit_lengths, logits.reshape(T, N * V))
    return probs, packed.reshape(N, T, K, 3)


# speedup vs baseline: 6.1926x; 1.4992x over previous
import functools

import jax
import jax.numpy as jnp
import numpy as np
from jax.experimental import pallas as pl
from jax.experimental.pallas import tpu as pltpu

K = 40
V = 128
TB = 512


def _bitonic_sort_desc(keys):
    rows = keys.shape[0]
    lane = jax.lax.broadcasted_iota(jnp.int32, (rows, V), 1)
    for k in (2, 4, 8, 16, 32, 64, 128):
        j = k // 2
        while j >= 1:
            is_lower = (lane & j) == 0
            partner = jnp.take_along_axis(keys, lane ^ j, axis=1)
            seg_desc = (lane & k) == 0
            keep_min = seg_desc != is_lower
            swap = (partner > keys) != keep_min
            keys = jnp.where(swap, partner, keys)
            j //= 2
    return keys


def _ctc_kernel(len_ref, x_ref, probs_ref, packed_ref, *, tb):
    n = pl.program_id(0)
    t0 = pl.program_id(1) * tb
    x = x_ref[...]

    m = jnp.max(x, axis=-1, keepdims=True)
    e = jnp.exp(x - m)
    s = jnp.sum(e, axis=-1, keepdims=True)
    p = e * (1.0 / s)

    lane = jax.lax.broadcasted_iota(jnp.int32, (tb, V), 1)
    pb = jax.lax.bitcast_convert_type(p, jnp.int32)
    key = ((pb + 64) & jnp.int32(~127)) | (127 - lane)

    skey = _bitonic_sort_desc(key)[:, :K]
    vals = jax.lax.bitcast_convert_type(skey & jnp.int32(~127), jnp.float32)
    idx = 127 - (skey & 127)

    t_row = t0 + jax.lax.broadcasted_iota(jnp.int32, (tb, 1), 0)
    mask = t_row < len_ref[n]

    probs_ref[0, :, :] = jnp.where(mask, vals, 0.0)

    selr = jax.lax.broadcasted_iota(jnp.int32, (K, 3 * K), 0)
    selc = jax.lax.broadcasted_iota(jnp.int32, (K, 3 * K), 1)
    sel = (selc == 3 * selr + 2).astype(jnp.float32)
    idx_rep = jax.lax.dot_general(
        idx.astype(jnp.float32),
        sel,
        (((1,), (0,)), ((), ())),
        preferred_element_type=jnp.float32,
    ).astype(jnp.int32)
    lane3 = jax.lax.broadcasted_iota(jnp.int32, (tb, 3 * K), 1) % 3
    interleaved = jnp.where(
        lane3 == 0, n, jnp.where(lane3 == 1, t_row, idx_rep)
    )
    packed_ref[0, :, :] = jnp.where(mask, interleaved, -1)


@jax.jit
def kernel(logits, logit_lengths):
    T, N, _ = logits.shape
    tb = TB
    grid = (N, T // tb)
    probs, packed = pl.pallas_call(
        functools.partial(_ctc_kernel, tb=tb),
        grid=grid,
        in_specs=[
            pl.BlockSpec(memory_space=pltpu.SMEM),
            pl.BlockSpec((tb, V), lambda n, t: (t, n)),
        ],
        out_specs=[
            pl.BlockSpec((1, tb, K), lambda n, t: (n, t, 0)),
            pl.BlockSpec((1, tb, 3 * K), lambda n, t: (n, t, 0)),
        ],
        out_shape=[
            jax.ShapeDtypeStruct((N, T, K), jnp.float32),
            jax.ShapeDtypeStruct((N, T, 3 * K), jnp.int32),
        ],
        compiler_params=pltpu.CompilerParams(
            dimension_semantics=("parallel", "parallel"),
        ),
    )(logit_lengths, logits.reshape(T, N * V))
    return probs, packed.reshape(N, T, K, 3)


# speedup vs baseline: 7.7028x; 1.2439x over previous
import functools

import jax
import jax.numpy as jnp
from jax.experimental import pallas as pl
from jax.experimental.pallas import tpu as pltpu

K = 40
V = 128
TB = 2048
NG = 8


def _bitonic_sort_desc(keys):
    rows = keys.shape[0]
    lane = jax.lax.broadcasted_iota(jnp.int32, (rows, V), 1)
    for k in (2, 4, 8, 16, 32, 64, 128):
        j = k // 2
        while j >= 1:
            is_lower = (lane & j) == 0
            partner = jnp.take_along_axis(keys, lane ^ j, axis=1)
            seg_desc = (lane & k) == 0
            keep_min = seg_desc != is_lower
            swap = (partner > keys) != keep_min
            keys = jnp.where(swap, partner, keys)
            j //= 2
    return keys


def _ctc_kernel(len_ref, x_ref, probs_ref, idx_ref, *, tb):
    ng = pl.program_id(0)
    t0 = pl.program_id(1) * tb
    lane = jax.lax.broadcasted_iota(jnp.int32, (tb, V), 1)
    t_row = t0 + jax.lax.broadcasted_iota(jnp.int32, (tb, 1), 0)

    xt = jnp.transpose(x_ref[...], (1, 0, 2))
    for i in range(NG):
        n = ng * NG + i
        x = xt[i]

        m = jnp.max(x, axis=-1, keepdims=True)
        e = jnp.exp(x - m)
        s = jnp.sum(e, axis=-1, keepdims=True)
        p = e * (1.0 / s)

        pb = jax.lax.bitcast_convert_type(p, jnp.int32)
        key = ((pb + 64) & jnp.int32(~127)) | (127 - lane)

        skey = _bitonic_sort_desc(key)[:, :K]
        vals = jax.lax.bitcast_convert_type(skey & jnp.int32(~127), jnp.float32)
        idx = 127 - (skey & 127)

        mask = t_row < len_ref[n]
        probs_ref[i, :, :] = jnp.where(mask, vals, 0.0)
        idx_ref[i, :, :] = jnp.where(mask, idx, -1)


@jax.jit
def kernel(logits, logit_lengths):
    T, N, _ = logits.shape
    tb = TB
    grid = (N // NG, T // tb)
    probs, idxs = pl.pallas_call(
        functools.partial(_ctc_kernel, tb=tb),
        grid=grid,
        in_specs=[
            pl.BlockSpec(memory_space=pltpu.SMEM),
            pl.BlockSpec((tb, NG, V), lambda g, t: (t, g, 0)),
        ],
        out_specs=[
            pl.BlockSpec((NG, tb, K), lambda g, t: (g, t, 0)),
            pl.BlockSpec((NG, tb, K), lambda g, t: (g, t, 0)),
        ],
        out_shape=[
            jax.ShapeDtypeStruct((N, T, K), jnp.float32),
            jax.ShapeDtypeStruct((N, T, K), jnp.int32),
        ],
        compiler_params=pltpu.CompilerParams(
            dimension_semantics=("parallel", "parallel"),
        ),
    )(logit_lengths, logits)
    t_iota = jax.lax.broadcasted_iota(jnp.int32, (N, T, K), 1)
    n_iota = jax.lax.broadcasted_iota(jnp.int32, (N, T, K), 0)
    mask = t_iota < logit_lengths.reshape(-1, 1, 1)
    neg1 = jnp.int32(-1)
    valid = jnp.stack(
        (
            jnp.where(mask, n_iota, neg1),
            jnp.where(mask, t_iota, neg1),
            idxs,
        ),
        axis=-1,
    )
    return probs, valid
